# baseline (speedup 1.0000x reference)
import functools

import jax
import jax.numpy as jnp
from jax.experimental import pallas as pl

DEPTH = 8
G = 8
GD = 48
D = G * GD


def _tpq_kernel(x_ref, wd2x_ref, wdt_ref, v0sum_ref, cvec_ref, a2_ref,
                xq_ref, idx_ref, acc_ref, *, blk):
    x = x_ref[...]
    dn = (((1,), (0,)), ((), ()))
    bitfs = []
    idxf = jnp.zeros((blk, G), jnp.float32)
    for k in range(DEPTH):
        e = jax.lax.dot_general(
            x, wd2x_ref[k], dn,
            precision=jax.lax.Precision.HIGHEST,
            preferred_element_type=jnp.float32)
        for j in range(k):
            r = j * DEPTH + k
            e = e - bitfs[j] * a2_ref[r:r + 1, :]
        bit = e > cvec_ref[k:k + 1, :]
        bf = bit.astype(jnp.float32)
        bitfs.append(bf)
        idxf = idxf + bf * float(1 << k)
    xq = jax.lax.dot_general(bitfs[0], wdt_ref[0], dn,
                             preferred_element_type=jnp.float32)
    for k in range(1, DEPTH):
        xq = xq + jax.lax.dot_general(bitfs[k], wdt_ref[k], dn,
                                      preferred_element_type=jnp.float32)
    xq = xq + v0sum_ref[...]
    t = xq - x
    xq_ref[...] = x + t
    idx_ref[...] = idxf.astype(jnp.int32)
    p = jnp.sum(t * t)
    i = pl.program_id(0)

    @pl.when(i == 0)
    def _():
        acc_ref[...] = jnp.full((8, 128), p, jnp.float32)

    @pl.when(i > 0)
    def _():
        acc_ref[...] = acc_ref[...] + p


def kernel(x, levels):
    B, T, _ = x.shape
    x2 = x.reshape(B * T, D)
    n = B * T

    lv = levels.astype(jnp.float32)
    v0 = lv[:, :, 0, :]
    v1 = lv[:, :, 1, :]
    wd = v1 - v0
    eye = jnp.eye(G, dtype=jnp.float32)
    wd2x = jnp.einsum('gkd,gh->kgdh', 2.0 * wd, eye).reshape(DEPTH, D, G)
    wdt = jnp.einsum('gkd,hg->khgd', wd, eye).reshape(DEPTH, G, D)
    v0sum = jnp.sum(v0, axis=1).reshape(1, D)
    thr0 = jnp.sum(v1 * v1 - v0 * v0, axis=-1)
    p_jk = jnp.einsum('gjd,gkd->gjk', v0, wd)
    jlt = (jnp.arange(DEPTH)[:, None] < jnp.arange(DEPTH)[None, :])
    c = thr0 + 2.0 * jnp.sum(p_jk * jlt[None], axis=1)
    cvec = c.T
    a_jk = jnp.einsum('gjd,gkd->gjk', wd, wd)
    a2 = 2.0 * jnp.transpose(a_jk, (1, 2, 0)).reshape(DEPTH * DEPTH, G)

    blk = 512
    grid = n // blk
    xq2, idx2, acc = pl.pallas_call(
        functools.partial(_tpq_kernel, blk=blk),
        grid=(grid,),
        in_specs=[
            pl.BlockSpec((blk, D), lambda i: (i, 0)),
            pl.BlockSpec((DEPTH, D, G), lambda i: (0, 0, 0)),
            pl.BlockSpec((DEPTH, G, D), lambda i: (0, 0, 0)),
            pl.BlockSpec((1, D), lambda i: (0, 0)),
            pl.BlockSpec((DEPTH, G), lambda i: (0, 0)),
            pl.BlockSpec((DEPTH * DEPTH, G), lambda i: (0, 0)),
        ],
        out_specs=[
            pl.BlockSpec((blk, D), lambda i: (i, 0)),
            pl.BlockSpec((blk, G), lambda i: (i, 0)),
            pl.BlockSpec((8, 128), lambda i: (0, 0)),
        ],
        out_shape=[
            jax.ShapeDtypeStruct((n, D), jnp.float32),
            jax.ShapeDtypeStruct((n, G), jnp.int32),
            jax.ShapeDtypeStruct((8, 128), jnp.float32),
        ],
    )(x2, wd2x, wdt, v0sum, cvec, a2)

    total_loss = (2.0 / (B * T * GD)) * acc[0, 0]
    return (xq2.reshape(B, T, D), total_loss, idx2.reshape(B, T, G))


# speedup vs baseline: 3.5249x; 3.5249x over previous
import functools

import jax
import jax.numpy as jnp
from jax.experimental import pallas as pl

DEPTH = 8
G = 8
GD = 48
D = G * GD
GK = G * DEPTH


def _tpq_kernel(x_ref, wd2x_ref, wdt_ref, v0sum_ref, cvec_ref, a2_ref,
                xq_ref, idx_ref, acc_ref, *, blk):
    x = x_ref[...]
    dn = (((1,), (0,)), ((), ()))
    s = jax.lax.dot_general(
        x, wd2x_ref[...], dn,
        precision=jax.lax.Precision.HIGHEST,
        preferred_element_type=jnp.float32)
    st = jnp.transpose(s, (1, 0))
    bitfs = []
    idxf = jnp.zeros((G, blk), jnp.float32)
    for k in range(DEPTH):
        e = st[8 * k:8 * k + 8, :]
        for j in range(k):
            e = e - bitfs[j] * a2_ref[j * DEPTH + k]
        bit = e > cvec_ref[k]
        bf = bit.astype(jnp.float32)
        bitfs.append(bf)
        idxf = idxf + bf * float(1 << k)
    bits64t = jnp.concatenate(bitfs, axis=0)
    bits64 = jnp.transpose(bits64t, (1, 0))
    xq = jax.lax.dot_general(bits64, wdt_ref[...], dn,
                             preferred_element_type=jnp.float32)
    xq = xq + v0sum_ref[...]
    t = xq - x
    xq_ref[...] = x + t
    idx_ref[...] = jnp.transpose(idxf, (1, 0)).astype(jnp.int32)
    p = jnp.sum(t * t)
    i = pl.program_id(0)

    @pl.when(i == 0)
    def _():
        acc_ref[...] = jnp.full((8, 128), p, jnp.float32)

    @pl.when(i > 0)
    def _():
        acc_ref[...] = acc_ref[...] + p


def kernel(x, levels):
    B, T, _ = x.shape
    x2 = x.reshape(B * T, D)
    n = B * T

    lv = levels.astype(jnp.float32)
    v0 = lv[:, :, 0, :]
    v1 = lv[:, :, 1, :]
    wd = v1 - v0
    eye = jnp.eye(G, dtype=jnp.float32)
    wd2x = jnp.einsum('gkd,gh->gdkh', 2.0 * wd, eye).reshape(D, GK)
    wdt = jnp.einsum('gkd,hg->khgd', wd, eye).reshape(GK, D)
    v0sum = jnp.sum(v0, axis=1).reshape(1, D)
    thr0 = jnp.sum(v1 * v1 - v0 * v0, axis=-1)
    p_jk = jnp.einsum('gjd,gkd->gjk', v0, wd)
    jlt = (jnp.arange(DEPTH)[:, None] < jnp.arange(DEPTH)[None, :])
    c = thr0 + 2.0 * jnp.sum(p_jk * jlt[None], axis=1)
    cvec = c.T.reshape(DEPTH, G, 1)
    a_jk = jnp.einsum('gjd,gkd->gjk', wd, wd)
    a2 = 2.0 * jnp.transpose(a_jk, (1, 2, 0)).reshape(DEPTH * DEPTH, G, 1)

    blk = 512
    grid = n // blk
    xq2, idx2, acc = pl.pallas_call(
        functools.partial(_tpq_kernel, blk=blk),
        grid=(grid,),
        in_specs=[
            pl.BlockSpec((blk, D), lambda i: (i, 0)),
            pl.BlockSpec((D, GK), lambda i: (0, 0)),
            pl.BlockSpec((GK, D), lambda i: (0, 0)),
            pl.BlockSpec((1, D), lambda i: (0, 0)),
            pl.BlockSpec((DEPTH, G, 1), lambda i: (0, 0, 0)),
            pl.BlockSpec((DEPTH * DEPTH, G, 1), lambda i: (0, 0, 0)),
        ],
        out_specs=[
            pl.BlockSpec((blk, D), lambda i: (i, 0)),
            pl.BlockSpec((blk, G), lambda i: (i, 0)),
            pl.BlockSpec((8, 128), lambda i: (0, 0)),
        ],
        out_shape=[
            jax.ShapeDtypeStruct((n, D), jnp.float32),
            jax.ShapeDtypeStruct((n, G), jnp.int32),
            jax.ShapeDtypeStruct((8, 128), jnp.float32),
        ],
    )(x2, wd2x, wdt, v0sum, cvec, a2)

    total_loss = (2.0 / (B * T * GD)) * acc[0, 0]
    return (xq2.reshape(B, T, D), total_loss, idx2.reshape(B, T, G))


# speedup vs baseline: 4.1686x; 1.1826x over previous
import functools

import jax
import jax.numpy as jnp
from jax.experimental import pallas as pl

DEPTH = 8
G = 8
GD = 48
D = G * GD
GK = G * DEPTH


def _tpq_kernel(x_ref, wd2x_ref, wdt_ref, v0sum_ref, cvec_ref, a2_ref,
                xq_ref, idx_ref, acc_ref, *, blk):
    x = x_ref[...]
    dn = (((1,), (0,)), ((), ()))
    s = jax.lax.dot_general(
        x, wd2x_ref[...], dn,
        precision=jax.lax.Precision.HIGHEST,
        preferred_element_type=jnp.float32)
    st = jnp.transpose(s, (1, 0))
    bitfs = []
    idxf = jnp.zeros((G, blk), jnp.float32)
    for k in range(DEPTH):
        e = st[8 * k:8 * k + 8, :]
        for j in range(k):
            e = e - bitfs[j] * a2_ref[j * DEPTH + k]
        bit = e > cvec_ref[k]
        bf = bit.astype(jnp.float32)
        bitfs.append(bf)
        idxf = idxf + bf * float(1 << k)
    bits64t = jnp.concatenate(bitfs, axis=0)
    bits64 = jnp.transpose(bits64t, (1, 0))
    xq = jax.lax.dot_general(bits64, wdt_ref[...], dn,
                             preferred_element_type=jnp.float32)
    xq = xq + v0sum_ref[...]
    t = xq - x
    xq_ref[...] = x + t
    idx_ref[...] = jnp.transpose(idxf, (1, 0)).astype(jnp.int32)
    p = jnp.sum(t * t)
    i = pl.program_id(0)

    @pl.when(i == 0)
    def _():
        acc_ref[...] = jnp.full((8, 128), p, jnp.float32)

    @pl.when(i > 0)
    def _():
        acc_ref[...] = acc_ref[...] + p


def kernel(x, levels):
    B, T, _ = x.shape
    x2 = x.reshape(B * T, D)
    n = B * T

    lv = levels.astype(jnp.float32)
    v0 = lv[:, :, 0, :]
    v1 = lv[:, :, 1, :]
    wd = v1 - v0
    eye = jnp.eye(G, dtype=jnp.float32)
    wd2x = jnp.einsum('gkd,gh->gdkh', 2.0 * wd, eye).reshape(D, GK)
    wdt = jnp.einsum('gkd,hg->khgd', wd, eye).reshape(GK, D)
    v0sum = jnp.sum(v0, axis=1).reshape(1, D)
    thr0 = jnp.sum(v1 * v1 - v0 * v0, axis=-1)
    p_jk = jnp.einsum('gjd,gkd->gjk', v0, wd)
    jlt = (jnp.arange(DEPTH)[:, None] < jnp.arange(DEPTH)[None, :])
    c = thr0 + 2.0 * jnp.sum(p_jk * jlt[None], axis=1)
    cvec = c.T.reshape(DEPTH, G, 1)
    a_jk = jnp.einsum('gjd,gkd->gjk', wd, wd)
    a2 = 2.0 * jnp.transpose(a_jk, (1, 2, 0)).reshape(DEPTH * DEPTH, G, 1)

    blk = 1024
    grid = n // blk
    xq2, idx2, acc = pl.pallas_call(
        functools.partial(_tpq_kernel, blk=blk),
        grid=(grid,),
        in_specs=[
            pl.BlockSpec((blk, D), lambda i: (i, 0)),
            pl.BlockSpec((D, GK), lambda i: (0, 0)),
            pl.BlockSpec((GK, D), lambda i: (0, 0)),
            pl.BlockSpec((1, D), lambda i: (0, 0)),
            pl.BlockSpec((DEPTH, G, 1), lambda i: (0, 0, 0)),
            pl.BlockSpec((DEPTH * DEPTH, G, 1), lambda i: (0, 0, 0)),
        ],
        out_specs=[
            pl.BlockSpec((blk, D), lambda i: (i, 0)),
            pl.BlockSpec((blk, G), lambda i: (i, 0)),
            pl.BlockSpec((8, 128), lambda i: (0, 0)),
        ],
        out_shape=[
            jax.ShapeDtypeStruct((n, D), jnp.float32),
            jax.ShapeDtypeStruct((n, G), jnp.int32),
            jax.ShapeDtypeStruct((8, 128), jnp.float32),
        ],
    )(x2, wd2x, wdt, v0sum, cvec, a2)

    total_loss = (2.0 / (B * T * GD)) * acc[0, 0]
    return (xq2.reshape(B, T, D), total_loss, idx2.reshape(B, T, G))
